# baseline (speedup 1.0000x reference)
import functools

import jax
import jax.numpy as jnp
from jax.experimental import pallas as pl

N = 10000
D = 256
L = 4
NBLK = 10
BLK = N // NBLK


def _mm_body(h_ref, w_ref, b_ref, o_ref):
    o_ref[...] = (
        jnp.dot(h_ref[...], w_ref[...], preferred_element_type=jnp.float32)
        + b_ref[...]
    )


def _fused_matmul(h, wcat, bcat):
    kd = wcat.shape[1]
    return pl.pallas_call(
        _mm_body,
        grid=(NBLK,),
        in_specs=[
            pl.BlockSpec((BLK, D), lambda i: (i, 0)),
            pl.BlockSpec((D, kd), lambda i: (0, 0)),
            pl.BlockSpec((1, kd), lambda i: (0, 0)),
        ],
        out_specs=pl.BlockSpec((BLK, kd), lambda i: (i, 0)),
        out_shape=jax.ShapeDtypeStruct((N, kd), jnp.float32),
    )(h, wcat, bcat)


def _ln(v, g, b):
    mu = v.mean(axis=-1, keepdims=True)
    var = v.var(axis=-1, keepdims=True)
    return (v - mu) / jnp.sqrt(var + 1e-5) * g + b


def kernel(x, edge_index, target_indices, edge_list_0, edge_list_1, emb, W_root, b_root, W_bin, b_bin, W_ter, b_ter, ln_g, ln_b, W_int, b_int, W_mlp, b_mlp, ln_mlp_g, ln_mlp_b, W_out, b_out):
    def build_w(l):
        cols = [W_root[l]]
        cols += [W_bin[l, p][:D] for p in range(2)]
        cols += [W_bin[l, p][D:] for p in range(2)]
        for q in range(3):
            cols += [W_ter[l, p][q * D:(q + 1) * D] for p in range(3)]
        return jnp.concatenate(cols, axis=1)

    def build_b(l):
        z = jnp.zeros((D,), jnp.float32)
        parts = [b_root[l], b_bin[l, 0], b_bin[l, 1], z, z,
                 b_ter[l, 0], b_ter[l, 1], b_ter[l, 2]] + [z] * 6
        return jnp.concatenate(parts)[None, :]

    s0 = edge_list_0[:, 0]
    d0 = edge_list_0[:, 1]
    e1a, e1b, e1c = edge_list_1[:, 0], edge_list_1[:, 1], edge_list_1[:, 2]

    h = emb[jnp.ravel(x)]
    inter_t = [h[target_indices]]
    for l in range(L):
        P = _fused_matmul(h, build_w(l), build_b(l))
        agg = P[:, 0:D]
        U0, U1 = P[:, D:2 * D], P[:, 2 * D:3 * D]
        V0, V1 = P[:, 3 * D:4 * D], P[:, 4 * D:5 * D]
        agg = agg.at[s0].add(jax.nn.relu(U0[s0] + V0[d0]))
        agg = agg.at[d0].add(jax.nn.relu(U1[s0] + V1[d0]))
        for p in range(3):
            base = (5 + p) * D
            m = jax.nn.relu(P[:, base:base + D][e1a]
                            + P[:, base + 3 * D:base + 4 * D][e1b]
                            + P[:, base + 6 * D:base + 7 * D][e1c])
            agg = agg.at[[e1a, e1b, e1c][p]].add(m)
        h = jax.nn.relu(_ln(agg, ln_g[l], ln_b[l]))
        inter_t.append(h[target_indices])

    z = jnp.concatenate(inter_t, axis=1)
    z = z @ W_int + b_int
    for j in range(2):
        z = z @ W_mlp[j] + b_mlp[j]
        z = jax.nn.relu(_ln(z, ln_mlp_g[j], ln_mlp_b[j]))
    return z @ W_out + b_out


# speedup vs baseline: 1.8671x; 1.8671x over previous
import functools

import jax
import jax.numpy as jnp
from jax import lax
from jax.experimental import pallas as pl
from jax.experimental.pallas import tpu as pltpu
from jax.experimental.pallas import tpu_sc as plsc

D = 256
H = 128
N = 10000
NP = 10240
NBLK = 16
BLK = NP // NBLK
E0, E1 = 50000, 20000
E0P = 16 * 3328
E1P = 16 * 1280
NB0 = 3328 // 128
NB1 = 1280 // 128
CH = 128
PAD = N
KD = 14 * D
T = 1024

_MESH = plsc.VectorSubcoreMesh(core_axis_name="c", subcore_axis_name="s")


def _prep_outs(with_h):
    outs = []
    if with_h:
        outs.append(jax.ShapeDtypeStruct((NP, D), jnp.float32))
    outs += [jax.ShapeDtypeStruct((2, NP, H), jnp.float32)] * 14
    return tuple(outs)


def _write_products(P, outs):
    for i, ref in enumerate(outs):
        ref[0] = P[:, i * H:(i + 1) * H]
        ref[1] = P[:, 1792 + i * H:1792 + (i + 1) * H]


def _prep0_body(h_ref, w_ref, b_ref, *outs):
    P = jnp.dot(h_ref[...], w_ref[...],
                preferred_element_type=jnp.float32) + b_ref[...]
    _write_products(P, outs)


def _prep_body(agg_ref, w_ref, b_ref, g_ref, bb_ref, hout_ref, *outs):
    a = jnp.concatenate([agg_ref[0], agg_ref[1]], axis=1)
    mu = jnp.mean(a, axis=1, keepdims=True)
    var = jnp.mean((a - mu) ** 2, axis=1, keepdims=True)
    h = (a - mu) * lax.rsqrt(var + 1e-5) * g_ref[...] + bb_ref[...]
    h = jnp.maximum(h, 0.0)
    hout_ref[...] = h
    P = jnp.dot(h, w_ref[...],
                preferred_element_type=jnp.float32) + b_ref[...]
    _write_products(P, outs)


def _prod_specs():
    return [pl.BlockSpec((2, BLK, H), lambda i: (0, i, 0))] * 14


def _prep0(h, w, b):
    return pl.pallas_call(
        _prep0_body,
        grid=(NBLK,),
        in_specs=[
            pl.BlockSpec((BLK, D), lambda i: (i, 0)),
            pl.BlockSpec((D, KD), lambda i: (0, 0)),
            pl.BlockSpec((1, KD), lambda i: (0, 0)),
        ],
        out_specs=_prod_specs(),
        out_shape=_prep_outs(False),
    )(h, w, b)


def _prep(agg, w, b, g, bb):
    return pl.pallas_call(
        _prep_body,
        grid=(NBLK,),
        in_specs=[
            pl.BlockSpec((2, BLK, H), lambda i: (0, i, 0)),
            pl.BlockSpec((D, KD), lambda i: (0, 0)),
            pl.BlockSpec((1, KD), lambda i: (0, 0)),
            pl.BlockSpec((1, D), lambda i: (0, 0)),
            pl.BlockSpec((1, D), lambda i: (0, 0)),
        ],
        out_specs=[pl.BlockSpec((BLK, D), lambda i: (i, 0))] + _prod_specs(),
        out_shape=_prep_outs(True),
    )(agg, w, b, g, bb)


@functools.partial(
    pl.kernel,
    out_type=jax.ShapeDtypeStruct((2, NP, H), jnp.float32),
    mesh=_MESH,
    scratch_types=[
        pltpu.VMEM((CH, H), jnp.float32),
        pltpu.VMEM((CH, H), jnp.float32),
        pltpu.VMEM((CH,), jnp.int32),
        pltpu.VMEM((CH,), jnp.int32),
        pltpu.VMEM((CH,), jnp.int32),
        pltpu.VMEM_SHARED((NP, H), jnp.float32),
        pltpu.SemaphoreType.DMA,
        pltpu.SemaphoreType.DMA,
    ],
)
def _edge(root_r, u0, u1, v0, v1, t00, t10, t20, t01, t11, t21, t02, t12, t22,
          e0s_r, e0d_r, e1a_r, e1b_r, e1c_r, out_r,
          A, B, iA, iB, iC, agg, semA, semB):
    c = lax.axis_index("c")
    s = lax.axis_index("s")
    r0 = s * BLK

    def relu_add(dst, src):
        def row(r, cr):
            for g in range(H // 16):
                o = g * 16
                dst[r, pl.ds(o, 16)] = jnp.maximum(
                    dst[r, pl.ds(o, 16)] + src[r, pl.ds(o, 16)], 0.0)
            return cr
        lax.fori_loop(0, CH, row, 0)

    def plain_add(dst, src):
        def row(r, cr):
            for g in range(H // 16):
                o = g * 16
                dst[r, pl.ds(o, 16)] = (dst[r, pl.ds(o, 16)]
                                        + src[r, pl.ds(o, 16)])
            return cr
        lax.fori_loop(0, CH, row, 0)

    for k2 in range(BLK // CH):
        pltpu.sync_copy(root_r.at[c, pl.ds(r0 + k2 * CH, CH)], A)
        pltpu.sync_copy(A, agg.at[pl.ds(r0 + k2 * CH, CH)])
    plsc.subcore_barrier()

    usrc = ((u0, v0), (u1, v1))

    def bin_chunk(k2, carry):
        base = s * (NB0 * CH) + k2 * CH
        pltpu.sync_copy(e0s_r.at[pl.ds(base, CH)], iA)
        pltpu.sync_copy(e0d_r.at[pl.ds(base, CH)], iB)
        for p in range(2):
            up, vp = usrc[p]
            cpA = pltpu.async_copy(up.at[c].at[iA], A, semA)
            cpB = pltpu.async_copy(vp.at[c].at[iB], B, semB)
            cpA.wait()
            cpB.wait()
            relu_add(A, B)
            pltpu.sync_copy(A, agg.at[(iA, iB)[p]], add=True)
        return carry
    lax.fori_loop(0, NB0, bin_chunk, 0)

    tsrc = ((t00, t01, t02), (t10, t11, t12), (t20, t21, t22))

    def ter_chunk(k2, carry):
        base = s * (NB1 * CH) + k2 * CH
        pltpu.sync_copy(e1a_r.at[pl.ds(base, CH)], iA)
        pltpu.sync_copy(e1b_r.at[pl.ds(base, CH)], iB)
        pltpu.sync_copy(e1c_r.at[pl.ds(base, CH)], iC)
        for p in range(3):
            ta, tb, tc2 = tsrc[p]
            cpA = pltpu.async_copy(ta.at[c].at[iA], A, semA)
            cpB = pltpu.async_copy(tb.at[c].at[iB], B, semB)
            cpA.wait()
            cpB.wait()
            plain_add(A, B)
            pltpu.async_copy(tc2.at[c].at[iC], B, semA).wait()
            relu_add(A, B)
            pltpu.sync_copy(A, agg.at[(iA, iB, iC)[p]], add=True)
        return carry
    lax.fori_loop(0, NB1, ter_chunk, 0)

    plsc.subcore_barrier()
    for k2 in range(BLK // CH):
        pltpu.sync_copy(agg.at[pl.ds(r0 + k2 * CH, CH)], A)
        pltpu.sync_copy(A, out_r.at[c, pl.ds(r0 + k2 * CH, CH)])


@functools.partial(
    pl.kernel,
    out_type=jax.ShapeDtypeStruct((NP, D), jnp.float32),
    mesh=_MESH,
    scratch_types=[
        pltpu.VMEM((CH,), jnp.int32),
        pltpu.VMEM((CH, D), jnp.float32),
        pltpu.SemaphoreType.DMA,
    ],
)
def _emb_gather(xp_r, emb_r, out_r, ix, buf, sem):
    c = lax.axis_index("c")
    s = lax.axis_index("s")
    w = s * 2 + c
    base = w * (NP // 32)
    for k2 in range(2):
        pltpu.sync_copy(xp_r.at[pl.ds(base + k2 * CH, CH)], ix)
        pltpu.async_copy(emb_r.at[ix], buf, sem).wait()
        pltpu.sync_copy(buf, out_r.at[pl.ds(base + k2 * CH, CH)])
    pltpu.sync_copy(xp_r.at[pl.ds(base + 2 * CH, 64)], ix.at[pl.ds(0, 64)])
    pltpu.async_copy(emb_r.at[ix.at[pl.ds(0, 64)]],
                     buf.at[pl.ds(0, 64)], sem).wait()
    pltpu.sync_copy(buf.at[pl.ds(0, 64)], out_r.at[pl.ds(base + 2 * CH, 64)])


@functools.partial(
    pl.kernel,
    out_type=tuple([jax.ShapeDtypeStruct((T, D), jnp.float32)] * 4
                   + [jax.ShapeDtypeStruct((2, T, H), jnp.float32)]),
    mesh=_MESH,
    scratch_types=[
        pltpu.VMEM((32,), jnp.int32),
        pltpu.VMEM((32, D), jnp.float32),
        pltpu.VMEM((32, H), jnp.float32),
        pltpu.SemaphoreType.DMA,
    ],
)
def _head_gather(tgt_r, h0_r, h1_r, h2_r, h3_r, agg_r,
                 z0, z1, z2, z3, z4, ix, buf, bufh, sem):
    c = lax.axis_index("c")
    s = lax.axis_index("s")
    base = (s * 2 + c) * 32
    pltpu.sync_copy(tgt_r.at[pl.ds(base, 32)], ix)
    for src, dst in ((h0_r, z0), (h1_r, z1), (h2_r, z2), (h3_r, z3)):
        pltpu.async_copy(src.at[ix], buf, sem).wait()
        pltpu.sync_copy(buf, dst.at[pl.ds(base, 32)])
    for c2 in range(2):
        pltpu.async_copy(agg_r.at[c2].at[ix], bufh, sem).wait()
        pltpu.sync_copy(bufh, z4.at[c2, pl.ds(base, 32)])


def _head_body(z0, z1, z2, z3, z4, wi_ref, bi_ref, wm_ref, bm_ref,
               lg_ref, lb_ref, g3_ref, b3_ref, wo_ref, bo_ref, out_ref):
    a = jnp.concatenate([z4[0], z4[1]], axis=1)
    mu = jnp.mean(a, axis=1, keepdims=True)
    var = jnp.mean((a - mu) ** 2, axis=1, keepdims=True)
    h4 = jnp.maximum(
        (a - mu) * lax.rsqrt(var + 1e-5) * g3_ref[...] + b3_ref[...], 0.0)
    wi = wi_ref[...]
    z = (jnp.dot(z0[...], wi[0:D], preferred_element_type=jnp.float32)
         + jnp.dot(z1[...], wi[D:2 * D], preferred_element_type=jnp.float32)
         + jnp.dot(z2[...], wi[2 * D:3 * D], preferred_element_type=jnp.float32)
         + jnp.dot(z3[...], wi[3 * D:4 * D], preferred_element_type=jnp.float32)
         + jnp.dot(h4, wi[4 * D:5 * D], preferred_element_type=jnp.float32)
         + bi_ref[...])
    for j in range(2):
        z = jnp.dot(z, wm_ref[j], preferred_element_type=jnp.float32) \
            + bm_ref[pl.ds(j, 1)]
        mu = jnp.mean(z, axis=1, keepdims=True)
        var = jnp.mean((z - mu) ** 2, axis=1, keepdims=True)
        z = jnp.maximum((z - mu) * lax.rsqrt(var + 1e-5)
                        * lg_ref[pl.ds(j, 1)] + lb_ref[pl.ds(j, 1)], 0.0)
    out_ref[...] = jnp.dot(z, wo_ref[...],
                           preferred_element_type=jnp.float32) + bo_ref[...]


def _head(z0, z1, z2, z3, z4, wi, bi, wm, bm, lg, lb, g3, b3, wo, bo):
    return pl.pallas_call(
        _head_body,
        out_shape=jax.ShapeDtypeStruct((T, H), jnp.float32),
    )(z0, z1, z2, z3, z4, wi, bi, wm, bm, lg, lb, g3, b3, wo, bo)


def kernel(x, edge_index, target_indices, edge_list_0, edge_list_1, emb,
           W_root, b_root, W_bin, b_bin, W_ter, b_ter, ln_g, ln_b,
           W_int, b_int, W_mlp, b_mlp, ln_mlp_g, ln_mlp_b, W_out, b_out):
    f32 = jnp.float32
    i32 = jnp.int32
    L = 4

    xp = jnp.zeros((NP,), i32).at[:N].set(x.astype(i32))
    e0s = jnp.full((E0P,), PAD, i32).at[:E0].set(edge_list_0[:, 0].astype(i32))
    e0d = jnp.full((E0P,), PAD, i32).at[:E0].set(edge_list_0[:, 1].astype(i32))
    e1a = jnp.full((E1P,), PAD, i32).at[:E1].set(edge_list_1[:, 0].astype(i32))
    e1b = jnp.full((E1P,), PAD, i32).at[:E1].set(edge_list_1[:, 1].astype(i32))
    e1c = jnp.full((E1P,), PAD, i32).at[:E1].set(edge_list_1[:, 2].astype(i32))
    tgt = target_indices.astype(i32)

    def build_wb(l):
        cols = ([W_root[l]]
                + [W_bin[l, p][:D] for p in range(2)]
                + [W_bin[l, p][D:] for p in range(2)])
        for q in range(3):
            cols += [W_ter[l, p][q * D:(q + 1) * D] for p in range(3)]
        Wf = jnp.concatenate(cols, axis=1)
        Wf = Wf.reshape(D, 14, 2, H).transpose(0, 2, 1, 3).reshape(D, KD)
        z = jnp.zeros((D,), f32)
        parts = [b_root[l], b_bin[l, 0], b_bin[l, 1], z, z,
                 b_ter[l, 0], b_ter[l, 1], b_ter[l, 2]] + [z] * 6
        bf = jnp.concatenate(parts).reshape(14, 2, H)
        bf = bf.transpose(1, 0, 2).reshape(1, KD)
        return Wf, bf

    h0 = _emb_gather(xp, emb)

    hs = [h0]
    agg = None
    for l in range(L):
        Wf, bf = build_wb(l)
        if l == 0:
            prods = _prep0(h0, Wf, bf)
        else:
            hout, *prods = _prep(agg, Wf, bf,
                                 ln_g[l - 1].reshape(1, D),
                                 ln_b[l - 1].reshape(1, D))
            hs.append(hout)
        agg = _edge(*prods, e0s, e0d, e1a, e1b, e1c)

    z0, z1, z2, z3, z4 = _head_gather(tgt, hs[0], hs[1], hs[2], hs[3], agg)

    wo = jnp.zeros((D, H), f32).at[:, :1].set(W_out)
    bo = jnp.zeros((1, H), f32).at[:, :1].set(b_out[None, :])
    out = _head(z0, z1, z2, z3, z4,
                W_int, b_int.reshape(1, D), W_mlp, b_mlp,
                ln_mlp_g, ln_mlp_b,
                ln_g[L - 1].reshape(1, D), ln_b[L - 1].reshape(1, D),
                wo, bo)
    return out[:, :1]
